# baseline (speedup 1.0000x reference)
import functools

import jax
import jax.numpy as jnp
from jax import lax
from jax.experimental import pallas as pl
from jax.experimental.pallas import tpu as pltpu
from jax.experimental.pallas import tpu_sc as plsc

N = 10000
E = 320000
F = 128
OUT = 3

_NC = 2
_NS = 16
_NW = _NC * _NS
_CH = 128
_NCHUNK = E // _CH
_JMAX = (_NCHUNK + _NW - 1) // _NW
_DP = F // 2

_SC_PARAMS = pltpu.CompilerParams(use_tc_tiling_on_sc=False)


@functools.lru_cache(maxsize=None)
def _sc_kernels():
    mesh = plsc.VectorSubcoreMesh(
        core_axis_name="c", subcore_axis_name="s", num_cores=_NC,
        num_subcores=_NS)

    @functools.partial(
        pl.kernel,
        mesh=mesh,
        out_type=jax.ShapeDtypeStruct((E, _DP), jnp.int32),
        compiler_params=_SC_PARAMS,
        scratch_types=[
            pltpu.VMEM((2, _CH), jnp.int32),
            pltpu.VMEM((_CH, _DP), jnp.int32),
            pltpu.VMEM((_CH, _DP), jnp.int32),
            pltpu.SemaphoreType.DMA,
            pltpu.SemaphoreType.DMA,
            pltpu.SemaphoreType.DMA,
            pltpu.SemaphoreType.DMA,
        ],
    )
    def sc_gather(x_hbm, src_hbm, out_hbm, idx_v, row0, row1,
                  g0, g1, w0, w1):
        wid = lax.axis_index("s") * _NC + lax.axis_index("c")
        rows = (row0, row1)
        gsems = (g0, g1)
        wsems = (w0, w1)

        def chunk_of(j):
            return wid + _NW * j

        def start_gather(j):
            b = j % 2
            chunk = chunk_of(j)

            @pl.when(chunk < _NCHUNK)
            def _():
                base = chunk * _CH
                pltpu.sync_copy(src_hbm.at[pl.ds(base, _CH)], idx_v.at[b])
                pltpu.async_copy(x_hbm.at[idx_v.at[b]], rows[b], gsems[b])

        def wait_gather(j):
            b = j % 2
            pltpu.make_async_copy(x_hbm.at[idx_v.at[b]], rows[b],
                                  gsems[b]).wait()

        def start_wb(j):
            b = j % 2
            base = chunk_of(j) * _CH
            pltpu.make_async_copy(rows[b], out_hbm.at[pl.ds(base, _CH)],
                                  wsems[b]).start()

        def wait_wb(j):
            b = j % 2
            base = chunk_of(j) * _CH
            pltpu.make_async_copy(rows[b], out_hbm.at[pl.ds(base, _CH)],
                                  wsems[b]).wait()

        start_gather(0)
        for j in range(_JMAX):
            if j + 1 < _JMAX:
                if j + 1 >= 2:
                    @pl.when(chunk_of(j - 1) < _NCHUNK)
                    def _(j=j):
                        wait_wb(j - 1)
                start_gather(j + 1)

            @pl.when(chunk_of(j) < _NCHUNK)
            def _(j=j):
                wait_gather(j)
                start_wb(j)

        for j in (_JMAX - 2, _JMAX - 1):
            @pl.when(chunk_of(j) < _NCHUNK)
            def _(j=j):
                wait_wb(j)

    @functools.partial(
        pl.kernel,
        mesh=mesh,
        out_type=jax.ShapeDtypeStruct((_NC, N, 8), jnp.float32),
        compiler_params=_SC_PARAMS,
        scratch_types=[
            pltpu.VMEM((_CH,), jnp.int32),
            pltpu.VMEM((_CH, 8), jnp.float32),
            pltpu.VMEM_SHARED((N, 8), jnp.float32),
        ],
    )
    def sc_scatter(msg_hbm, dst_hbm, zero_hbm, out_hbm, idx_v, msg_v, acc):
        cid = lax.axis_index("c")
        sid = lax.axis_index("s")
        wid = sid * _NC + cid

        @pl.when(sid == 0)
        def _():
            pltpu.sync_copy(zero_hbm, acc)

        plsc.subcore_barrier()

        def body(j, carry):
            chunk = wid + _NW * j

            @pl.when(chunk < _NCHUNK)
            def _():
                base = chunk * _CH
                pltpu.sync_copy(dst_hbm.at[pl.ds(base, _CH)], idx_v)
                pltpu.sync_copy(msg_hbm.at[pl.ds(base, _CH)], msg_v)
                pltpu.sync_copy(msg_v, acc.at[idx_v], add=True)

            return carry

        lax.fori_loop(0, _JMAX, body, 0)
        plsc.subcore_barrier()

        @pl.when(sid == 0)
        def _():
            pltpu.sync_copy(acc, out_hbm.at[cid])

    return sc_gather, sc_scatter


def _elu(v):
    return jnp.where(v > 0, v, jnp.exp(jnp.minimum(v, 0.0)) - 1.0)


def _msg_body(ea_ref, em_ref, xg_ref, w1t_ref, b1_ref, w2t_ref, b2_ref,
              a_ref, bb_ref, msg_ref):
    bf = jnp.bfloat16
    ea = ea_ref[...] * em_ref[...]
    h1 = jnp.dot(ea.astype(bf), w1t_ref[...],
                 preferred_element_type=jnp.float32) + b1_ref[...]
    h1 = _elu(h1)
    h2 = jnp.dot(h1.astype(bf), w2t_ref[...],
                 preferred_element_type=jnp.float32) + b2_ref[...]
    h2 = _elu(h2)
    xgi = xg_ref[...]
    xg_lo = lax.bitcast_convert_type(xgi << 16, jnp.float32)
    xg_hi = lax.bitcast_convert_type(xgi & jnp.int32(-65536), jnp.float32)
    xg = jnp.concatenate([xg_lo, xg_hi], axis=1)
    y = jnp.dot(xg.astype(bf), a_ref[...],
                preferred_element_type=jnp.float32)
    xb = jnp.dot(xg, bb_ref[...], preferred_element_type=jnp.float32)
    m0 = jnp.sum(h2 * y[:, 0 * F:1 * F], axis=1, keepdims=True)
    m1 = jnp.sum(h2 * y[:, 1 * F:2 * F], axis=1, keepdims=True)
    m2 = jnp.sum(h2 * y[:, 2 * F:3 * F], axis=1, keepdims=True)
    ones = jnp.ones_like(m0)
    zpad = jnp.zeros((m0.shape[0], 4), jnp.float32)
    msg_ref[...] = jnp.concatenate([m0, m1, m2, ones, zpad], axis=1) + xb


def _msg_call(ea, emf, xg, w1t, b1r, w2t, b2r, a, bb):
    bq = 512
    grid = (E // bq,)
    return pl.pallas_call(
        _msg_body,
        grid=grid,
        in_specs=[
            pl.BlockSpec((bq, F), lambda i: (i, 0)),
            pl.BlockSpec((bq, 1), lambda i: (i, 0)),
            pl.BlockSpec((bq, _DP), lambda i: (i, 0)),
            pl.BlockSpec((F, 256), lambda i: (0, 0)),
            pl.BlockSpec((1, 256), lambda i: (0, 0)),
            pl.BlockSpec((256, F), lambda i: (0, 0)),
            pl.BlockSpec((1, F), lambda i: (0, 0)),
            pl.BlockSpec((F, OUT * F), lambda i: (0, 0)),
            pl.BlockSpec((F, 8), lambda i: (0, 0)),
        ],
        out_specs=pl.BlockSpec((bq, 8), lambda i: (i, 0)),
        out_shape=jax.ShapeDtypeStruct((E, 8), jnp.float32),
        compiler_params=pltpu.CompilerParams(
            dimension_semantics=("arbitrary",)),
    )(ea, emf, xg, w1t, b1r, w2t, b2r, a, bb)


def _fin_body(x_ref, p_ref, wrt_ref, bias_ref, out_ref):
    s = (p_ref[0] + p_ref[1])[:, :4]
    cnt = s[:, 3:4]
    aggr = s / jnp.clip(cnt, 1.0, None)
    logits = jnp.dot(x_ref[...], wrt_ref[...],
                     preferred_element_type=jnp.float32) + aggr + bias_ref[...]
    m = jnp.max(logits, axis=0, keepdims=True)
    e = jnp.exp(logits - m)
    out_ref[...] = e / jnp.sum(e, axis=0, keepdims=True)


def _fin_call(xx, parts, wrtp, biasp):
    return pl.pallas_call(
        _fin_body,
        out_shape=jax.ShapeDtypeStruct((N, 4), jnp.float32),
    )(xx, parts, wrtp, biasp)


def kernel(x, edge_index, edge_attr, node_mask, edge_mask,
           W1, b1, W2, b2, W3, b3, W_root, bias):
    bf = jnp.bfloat16
    xx = jnp.where(node_mask[:, None], x, 0.0)
    edges = jnp.where(edge_mask[None, :], edge_index, 0)
    emf = edge_mask.astype(jnp.float32).reshape(E, 1)
    src = edges[0]
    dst = edges[1]

    xpack = lax.bitcast_convert_type(
        xx.astype(bf).reshape(N, _DP, 2), jnp.int32)

    w1t = W1.T.astype(bf)
    b1r = b1.reshape(1, 256)
    w2t = W2.T.astype(bf)
    b2r = b2.reshape(1, F)
    perm = jnp.concatenate(
        [jnp.arange(0, F, 2, dtype=jnp.int32),
         jnp.arange(1, F, 2, dtype=jnp.int32)])
    a = W3.reshape(F, OUT, F).reshape(F, OUT * F)[perm].astype(bf)
    bb = jnp.concatenate(
        [b3.reshape(F, OUT), jnp.zeros((F, 5), jnp.float32)], axis=1)[perm]
    wrtp = jnp.concatenate(
        [W_root.T, jnp.zeros((F, 1), jnp.float32)], axis=1)
    biasp = jnp.concatenate(
        [bias, jnp.zeros((1,), jnp.float32)]).reshape(1, 4)

    sc_gather, sc_scatter = _sc_kernels()
    xg = sc_gather(xpack, src)
    msg = _msg_call(edge_attr, emf, xg, w1t, b1r, w2t, b2r, a, bb)
    parts = sc_scatter(msg, dst, jnp.zeros((N, 8), jnp.float32))
    out4 = _fin_call(xx, parts, wrtp, biasp)
    return out4[:, :OUT]


# speedup vs baseline: 1.3477x; 1.3477x over previous
import functools

import jax
import jax.numpy as jnp
from jax import lax
from jax.experimental import pallas as pl
from jax.experimental.pallas import tpu as pltpu
from jax.experimental.pallas import tpu_sc as plsc

N = 10000
E = 320000
F = 128
OUT = 3

_NC = 2
_NS = 16
_NW = _NC * _NS
_CH = 128
_NCHUNK = E // _CH
_JMAX = (_NCHUNK + _NW - 1) // _NW

_SC_PARAMS = pltpu.CompilerParams(use_tc_tiling_on_sc=False)


@functools.lru_cache(maxsize=None)
def _sc_kernels():
    mesh = plsc.VectorSubcoreMesh(
        core_axis_name="c", subcore_axis_name="s", num_cores=_NC,
        num_subcores=_NS)

    @functools.partial(
        pl.kernel,
        mesh=mesh,
        out_type=jax.ShapeDtypeStruct((E, F), jnp.float32),
        compiler_params=_SC_PARAMS,
        scratch_types=[
            pltpu.VMEM((2, _CH), jnp.int32),
            pltpu.VMEM((_CH, F), jnp.float32),
            pltpu.VMEM((_CH, F), jnp.float32),
            pltpu.SemaphoreType.DMA,
            pltpu.SemaphoreType.DMA,
            pltpu.SemaphoreType.DMA,
            pltpu.SemaphoreType.DMA,
        ],
    )
    def sc_gather(x_hbm, src_hbm, out_hbm, idx_v, row0, row1,
                  g0, g1, w0, w1):
        wid = lax.axis_index("s") * _NC + lax.axis_index("c")
        rows = (row0, row1)
        gsems = (g0, g1)
        wsems = (w0, w1)

        def chunk_of(j):
            return wid + _NW * j

        def start_gather(j):
            b = j % 2
            chunk = chunk_of(j)

            @pl.when(chunk < _NCHUNK)
            def _():
                base = chunk * _CH
                pltpu.sync_copy(src_hbm.at[pl.ds(base, _CH)], idx_v.at[b])
                pltpu.async_copy(x_hbm.at[idx_v.at[b]], rows[b], gsems[b])

        def wait_gather(j):
            b = j % 2
            pltpu.make_async_copy(x_hbm.at[idx_v.at[b]], rows[b],
                                  gsems[b]).wait()

        def start_wb(j):
            b = j % 2
            base = chunk_of(j) * _CH
            pltpu.make_async_copy(rows[b], out_hbm.at[pl.ds(base, _CH)],
                                  wsems[b]).start()

        def wait_wb(j):
            b = j % 2
            base = chunk_of(j) * _CH
            pltpu.make_async_copy(rows[b], out_hbm.at[pl.ds(base, _CH)],
                                  wsems[b]).wait()

        start_gather(0)
        for j in range(_JMAX):
            if j + 1 < _JMAX:
                if j + 1 >= 2:
                    @pl.when(chunk_of(j - 1) < _NCHUNK)
                    def _(j=j):
                        wait_wb(j - 1)
                start_gather(j + 1)

            @pl.when(chunk_of(j) < _NCHUNK)
            def _(j=j):
                wait_gather(j)
                start_wb(j)

        for j in (_JMAX - 2, _JMAX - 1):
            @pl.when(chunk_of(j) < _NCHUNK)
            def _(j=j):
                wait_wb(j)

    @functools.partial(
        pl.kernel,
        mesh=mesh,
        out_type=jax.ShapeDtypeStruct((_NC, N, F), jnp.float32),
        compiler_params=_SC_PARAMS,
        scratch_types=[
            pltpu.VMEM((2, _CH), jnp.int32),
            pltpu.VMEM((_CH, F), jnp.float32),
            pltpu.VMEM((_CH, F), jnp.float32),
            pltpu.VMEM_SHARED((N, F), jnp.float32),
            pltpu.SemaphoreType.DMA,
            pltpu.SemaphoreType.DMA,
            pltpu.SemaphoreType.DMA,
            pltpu.SemaphoreType.DMA,
        ],
    )
    def sc_scatter(msg_hbm, dst_hbm, zero_hbm, out_hbm, idx_v, m0, m1,
                   acc, l0, l1, a0, a1):
        cid = lax.axis_index("c")
        sid = lax.axis_index("s")
        wid = sid * _NC + cid
        bufs = (m0, m1)
        lsems = (l0, l1)
        asems = (a0, a1)

        @pl.when(sid == 0)
        def _():
            pltpu.sync_copy(zero_hbm, acc)

        plsc.subcore_barrier()

        def chunk_of(j):
            return wid + _NW * j

        def start_load(j):
            b = j % 2
            chunk = chunk_of(j)

            @pl.when(chunk < _NCHUNK)
            def _():
                base = chunk * _CH
                pltpu.sync_copy(dst_hbm.at[pl.ds(base, _CH)], idx_v.at[b])
                pltpu.async_copy(msg_hbm.at[pl.ds(base, _CH)], bufs[b],
                                 lsems[b])

        def wait_load(j):
            b = j % 2
            base = chunk_of(j) * _CH
            pltpu.make_async_copy(msg_hbm.at[pl.ds(base, _CH)], bufs[b],
                                  lsems[b]).wait()

        def start_add(j):
            b = j % 2
            pltpu.make_async_copy(bufs[b], acc.at[idx_v.at[b]],
                                  asems[b]).start(add=True)

        def wait_add(j):
            b = j % 2
            pltpu.make_async_copy(bufs[b], acc.at[idx_v.at[b]],
                                  asems[b]).wait()

        start_load(0)
        for j in range(_JMAX):
            if j + 1 < _JMAX:
                if j + 1 >= 2:
                    @pl.when(chunk_of(j - 1) < _NCHUNK)
                    def _(j=j):
                        wait_add(j - 1)
                start_load(j + 1)

            @pl.when(chunk_of(j) < _NCHUNK)
            def _(j=j):
                wait_load(j)
                start_add(j)

        for j in (_JMAX - 2, _JMAX - 1):
            @pl.when(chunk_of(j) < _NCHUNK)
            def _(j=j):
                wait_add(j)

        plsc.subcore_barrier()

        @pl.when(sid == 0)
        def _():
            pltpu.sync_copy(acc, out_hbm.at[cid])

    return sc_gather, sc_scatter


def _elu(v):
    return jnp.where(v > 0, v, jnp.exp(jnp.minimum(v, 0.0)) - 1.0)


def _msg_body(ea_ref, xg_ref, w1t_ref, b1_ref, w2t_ref, b2_ref,
              a_ref, bb_ref, msg_ref):
    bf = jnp.bfloat16
    h1 = jnp.dot(ea_ref[...].astype(bf), w1t_ref[...],
                 preferred_element_type=jnp.float32) + b1_ref[...]
    h1 = _elu(h1)
    h2 = jnp.dot(h1.astype(bf), w2t_ref[...],
                 preferred_element_type=jnp.float32) + b2_ref[...]
    h2 = _elu(h2)
    xg = xg_ref[...]
    y = jnp.dot(xg.astype(bf), a_ref[...],
                preferred_element_type=jnp.float32)
    xb = jnp.dot(xg, bb_ref[...], preferred_element_type=jnp.float32)
    m0 = jnp.sum(h2 * y[:, 0 * F:1 * F], axis=1, keepdims=True)
    m1 = jnp.sum(h2 * y[:, 1 * F:2 * F], axis=1, keepdims=True)
    m2 = jnp.sum(h2 * y[:, 2 * F:3 * F], axis=1, keepdims=True)
    ones = jnp.ones_like(m0)
    zpad = jnp.zeros((m0.shape[0], F - 4), jnp.float32)
    msg_ref[...] = jnp.concatenate(
        [jnp.concatenate([m0, m1, m2, ones], axis=1) + xb, zpad], axis=1)


def _msg_call(ea, xg, w1t, b1r, w2t, b2r, a, bb):
    bq = 512
    grid = (E // bq,)
    return pl.pallas_call(
        _msg_body,
        grid=grid,
        in_specs=[
            pl.BlockSpec((bq, F), lambda i: (i, 0)),
            pl.BlockSpec((bq, F), lambda i: (i, 0)),
            pl.BlockSpec((F, 256), lambda i: (0, 0)),
            pl.BlockSpec((1, 256), lambda i: (0, 0)),
            pl.BlockSpec((256, F), lambda i: (0, 0)),
            pl.BlockSpec((1, F), lambda i: (0, 0)),
            pl.BlockSpec((F, OUT * F), lambda i: (0, 0)),
            pl.BlockSpec((F, 4), lambda i: (0, 0)),
        ],
        out_specs=pl.BlockSpec((bq, F), lambda i: (i, 0)),
        out_shape=jax.ShapeDtypeStruct((E, F), jnp.float32),
        compiler_params=pltpu.CompilerParams(
            dimension_semantics=("arbitrary",)),
    )(ea, xg, w1t, b1r, w2t, b2r, a, bb)


def _fin_body(x_ref, p_ref, wrt_ref, bias_ref, out_ref):
    s = (p_ref[0] + p_ref[1])[:, :4]
    cnt = s[:, 3:4]
    aggr = s / jnp.clip(cnt, 1.0, None)
    logits = jnp.dot(x_ref[...], wrt_ref[...],
                     preferred_element_type=jnp.float32) + aggr + bias_ref[...]
    m = jnp.max(logits, axis=0, keepdims=True)
    e = jnp.exp(logits - m)
    out_ref[...] = e / jnp.sum(e, axis=0, keepdims=True)


def _fin_call(xx, parts, wrtp, biasp):
    return pl.pallas_call(
        _fin_body,
        out_shape=jax.ShapeDtypeStruct((N, 4), jnp.float32),
    )(xx, parts, wrtp, biasp)


def kernel(x, edge_index, edge_attr, node_mask, edge_mask,
           W1, b1, W2, b2, W3, b3, W_root, bias):
    bf = jnp.bfloat16
    src = edge_index[0]
    dst = edge_index[1]

    w1t = W1.T.astype(bf)
    b1r = b1.reshape(1, 256)
    w2t = W2.T.astype(bf)
    b2r = b2.reshape(1, F)
    a = W3.reshape(F, OUT, F).reshape(F, OUT * F).astype(bf)
    bb = jnp.concatenate(
        [b3.reshape(F, OUT), jnp.zeros((F, 1), jnp.float32)], axis=1)
    wrtp = jnp.concatenate(
        [W_root.T, jnp.zeros((F, 1), jnp.float32)], axis=1)
    biasp = jnp.concatenate(
        [bias, jnp.zeros((1,), jnp.float32)]).reshape(1, 4)

    sc_gather, sc_scatter = _sc_kernels()
    xg = sc_gather(x, src)
    msg = _msg_call(edge_attr, xg, w1t, b1r, w2t, b2r, a, bb)
    parts = sc_scatter(msg, dst, jnp.zeros((N, F), jnp.float32))
    out4 = _fin_call(x, parts, wrtp, biasp)
    return out4[:, :OUT]
